# baseline (speedup 1.0000x reference)
import functools

import jax
import jax.numpy as jnp
from jax import lax
from jax.experimental import pallas as pl
from jax.experimental.pallas import tpu as pltpu
from jax.experimental.pallas import tpu_sc as plsc

DIM = 64
BATCH = 16384
NC = 2
NS = 16
NW = NC * NS
BPW = BATCH // NW
CHUNK = 128
NCHUNK = BPW // CHUNK
GROUPS = BPW // 16


def _dist_mult_body(h_hbm, r_hbm, t_hbm, ent_hbm, rel_hbm, out_hbm,
                    hidx, ridx, tidx, hrows, rrows, trows, outv, sem):
    wid = lax.axis_index("s") * NC + lax.axis_index("c")
    base_row = wid * NCHUNK
    base = wid * BPW

    pltpu.sync_copy(h_hbm.at[pl.ds(base_row, NCHUNK)], hidx)
    pltpu.sync_copy(r_hbm.at[pl.ds(base_row, NCHUNK)], ridx)
    pltpu.sync_copy(t_hbm.at[pl.ds(base_row, NCHUNK)], tidx)

    copies = []
    for j in range(NCHUNK):
        dst = pl.ds(j * CHUNK, CHUNK)
        copies.append(pltpu.async_copy(ent_hbm.at[hidx.at[j]], hrows.at[dst], sem))
        copies.append(pltpu.async_copy(ent_hbm.at[tidx.at[j]], trows.at[dst], sem))
        copies.append(pltpu.async_copy(rel_hbm.at[ridx.at[j]], rrows.at[dst], sem))
    for c in copies:
        c.wait()

    iot = lax.iota(jnp.int32, 16)

    dnums = lax.GatherDimensionNumbers(
        offset_dims=(), collapsed_slice_dims=(0,), start_index_map=(0,))

    def lane_gather(v, idx):
        return lax.gather(v, idx[:, None], dnums, slice_sizes=(1,),
                          mode=lax.GatherScatterMode.PROMISE_IN_BOUNDS)

    def hsum_splat(v):
        for sh in (8, 4, 2, 1):
            idx = (iot + sh) & 15
            v = v + lane_gather(v, idx)
        return v

    def group(g, carry):
        base16 = g * 16
        vec = jnp.zeros((16,), jnp.float32)
        for k16 in range(16):
            row = base16 + k16
            p = jnp.zeros((16,), jnp.float32)
            for k in range(DIM // 16):
                sl = pl.ds(k * 16, 16)
                p = p + hrows[row, sl] * rrows[row, sl] * trows[row, sl]
            s = hsum_splat(p)
            vec = jnp.where(iot == k16, s, vec)
        outv[pl.ds(pl.multiple_of(base16, 16), 16)] = vec
        return carry

    lax.fori_loop(0, GROUPS, group, 0)

    pltpu.sync_copy(outv, out_hbm.at[pl.ds(base, BPW)])


@jax.jit
def _dist_mult_sc(h2, r2, t2, ent_emb, rel_emb):
    mesh = plsc.VectorSubcoreMesh(core_axis_name="c", subcore_axis_name="s")
    kfn = functools.partial(
        pl.kernel,
        out_type=jax.ShapeDtypeStruct((BATCH,), jnp.float32),
        mesh=mesh,
        compiler_params=pltpu.CompilerParams(use_tc_tiling_on_sc=False),
        scratch_types=[
            pltpu.VMEM((NCHUNK, CHUNK), jnp.int32),
            pltpu.VMEM((NCHUNK, CHUNK), jnp.int32),
            pltpu.VMEM((NCHUNK, CHUNK), jnp.int32),
            pltpu.VMEM((BPW, DIM), jnp.float32),
            pltpu.VMEM((BPW, DIM), jnp.float32),
            pltpu.VMEM((BPW, DIM), jnp.float32),
            pltpu.VMEM((BPW,), jnp.float32),
            pltpu.SemaphoreType.DMA,
        ],
    )(_dist_mult_body)
    return kfn(h2, r2, t2, ent_emb, rel_emb)


def kernel(h, r, t, ent_emb, rel_emb):
    h2 = jnp.asarray(h, jnp.int32).reshape(NW * NCHUNK, CHUNK)
    r2 = jnp.asarray(r, jnp.int32).reshape(NW * NCHUNK, CHUNK)
    t2 = jnp.asarray(t, jnp.int32).reshape(NW * NCHUNK, CHUNK)
    return _dist_mult_sc(h2, r2, t2, ent_emb, rel_emb)


# speedup vs baseline: 1.6454x; 1.6454x over previous
import functools

import jax
import jax.numpy as jnp
from jax import lax
from jax.experimental import pallas as pl
from jax.experimental.pallas import tpu as pltpu
from jax.experimental.pallas import tpu_sc as plsc

DIM = 64
BATCH = 16384
NC = 2
NS = 16
NW = NC * NS
BPW = BATCH // NW
CHUNK = 128
NCHUNK = BPW // CHUNK
GPC = CHUNK // 16


def _dist_mult_body(h_hbm, r_hbm, t_hbm, ent_hbm, rel_hbm, out_hbm,
                    hidx, ridx, tidx, hbuf, rbuf, tbuf, outv,
                    sem0, sem1):
    wid = lax.axis_index("s") * NC + lax.axis_index("c")
    base = wid * BPW

    pltpu.sync_copy(h_hbm.at[pl.ds(wid, 1)], hidx)
    pltpu.sync_copy(r_hbm.at[pl.ds(wid, 1)], ridx)
    pltpu.sync_copy(t_hbm.at[pl.ds(wid, 1)], tidx)

    sems = (sem0, sem1)

    def issue_chunk(c, buf_slot):
        sem = sems[c % 2]
        hb, rb, tb = hbuf.at[buf_slot], rbuf.at[buf_slot], tbuf.at[buf_slot]

        def issue(j, _):
            off = c * CHUNK + j * 16
            hv = hidx[0, pl.ds(off, 16)]
            rv = ridx[0, pl.ds(off, 16)]
            tv = tidx[0, pl.ds(off, 16)]
            for k in range(16):
                dst = pl.ds(j * 16 + k, 1)
                pltpu.async_copy(ent_hbm.at[pl.ds(hv[k], 1)], hb.at[dst], sem)
                pltpu.async_copy(rel_hbm.at[pl.ds(rv[k], 1)], rb.at[dst], sem)
                pltpu.async_copy(ent_hbm.at[pl.ds(tv[k], 1)], tb.at[dst], sem)
            return 0

        lax.fori_loop(0, GPC, issue, 0)

    def drain_chunk(c, buf_slot):
        sem = sems[c % 2]
        dummy = ent_hbm.at[pl.ds(0, CHUNK)]
        pltpu.make_async_copy(dummy, hbuf.at[buf_slot], sem).wait()
        pltpu.make_async_copy(dummy, rbuf.at[buf_slot], sem).wait()
        pltpu.make_async_copy(dummy, tbuf.at[buf_slot], sem).wait()

    iot = lax.iota(jnp.int32, 16)
    dnums = lax.GatherDimensionNumbers(
        offset_dims=(), collapsed_slice_dims=(0,), start_index_map=(0,))

    def lane_gather(v, idx):
        return lax.gather(v, idx[:, None], dnums, slice_sizes=(1,),
                          mode=lax.GatherScatterMode.PROMISE_IN_BOUNDS)

    def hsum_splat(v):
        for sh in (8, 4, 2, 1):
            idx = (iot + sh) & 15
            v = v + lane_gather(v, idx)
        return v

    def compute_chunk(c, buf_slot):
        hb, rb, tb = hbuf.at[buf_slot], rbuf.at[buf_slot], tbuf.at[buf_slot]

        def group(g, carry):
            base16 = g * 16
            vec = jnp.zeros((16,), jnp.float32)
            for k16 in range(16):
                row = base16 + k16
                p = jnp.zeros((16,), jnp.float32)
                for k in range(DIM // 16):
                    sl = pl.ds(k * 16, 16)
                    p = p + hb[row, sl] * rb[row, sl] * tb[row, sl]
                s = hsum_splat(p)
                vec = jnp.where(iot == k16, s, vec)
            outv[pl.ds(pl.multiple_of(c * CHUNK + base16, 16), 16)] = vec
            return carry

        lax.fori_loop(0, GPC, group, 0)

    issue_chunk(0, 0)
    for c in range(NCHUNK):
        if c + 1 < NCHUNK:
            issue_chunk(c + 1, (c + 1) % 2)
        drain_chunk(c, c % 2)
        compute_chunk(c, c % 2)

    pltpu.sync_copy(outv, out_hbm.at[pl.ds(base, BPW)])


@jax.jit
def _dist_mult_sc(h2, r2, t2, ent_emb, rel_emb):
    mesh = plsc.VectorSubcoreMesh(core_axis_name="c", subcore_axis_name="s")
    kfn = functools.partial(
        pl.kernel,
        out_type=jax.ShapeDtypeStruct((BATCH,), jnp.float32),
        mesh=mesh,
        scratch_types=[
            pltpu.VMEM((1, BPW), jnp.int32),
            pltpu.VMEM((1, BPW), jnp.int32),
            pltpu.VMEM((1, BPW), jnp.int32),
            pltpu.VMEM((2, CHUNK, DIM), jnp.float32),
            pltpu.VMEM((2, CHUNK, DIM), jnp.float32),
            pltpu.VMEM((2, CHUNK, DIM), jnp.float32),
            pltpu.VMEM((BPW,), jnp.float32),
            pltpu.SemaphoreType.DMA,
            pltpu.SemaphoreType.DMA,
        ],
    )(_dist_mult_body)
    return kfn(h2, r2, t2, ent_emb, rel_emb)


def kernel(h, r, t, ent_emb, rel_emb):
    h2 = jnp.asarray(h, jnp.int32).reshape(NW, BPW)
    r2 = jnp.asarray(r, jnp.int32).reshape(NW, BPW)
    t2 = jnp.asarray(t, jnp.int32).reshape(NW, BPW)
    return _dist_mult_sc(h2, r2, t2, ent_emb, rel_emb)
